# Initial kernel scaffold; baseline (speedup 1.0000x reference)
import jax
import jax.numpy as jnp
from jax.experimental import pallas as pl


def kernel(x, gold):
    raise NotImplementedError("write your pallas kernel here")


# speedup vs baseline: 7.7390x; 7.7390x over previous
import functools

import jax
import jax.numpy as jnp
from jax import lax
from jax.experimental import pallas as pl

_B, _S, _V = 32, 32, 32768
_NEG = -3.0e38


def _loss_body(x_ref, gold_ref, out_ref):
    b = pl.program_id(0)
    xb = x_ref[0]
    y = gold_ref[0, 0]
    iota = lax.broadcasted_iota(jnp.int32, (_S, _V), 1)
    m1 = jnp.max(xb, axis=1)
    i1 = jnp.min(jnp.where(xb == m1[:, None], iota, _V), axis=1)
    m2 = jnp.max(jnp.where(iota == i1[:, None], _NEG, xb), axis=1)
    g = jnp.max(jnp.where(iota == y[:, None], xb, _NEG), axis=1)

    neg = jnp.where(i1 == y, m2, m1)
    loss = jnp.maximum(1.0 + neg - g, 0.0)
    loss = jnp.where(y != 0, loss, 0.0)
    ltot = jnp.sum(loss)
    anynz = jnp.any(y != 0)
    contrib = jnp.where(anynz, ltot, 0.0) * (1.0 / _B)

    contrib2 = contrib[None, None]

    @pl.when(b == 0)
    def _():
        out_ref[:, :] = contrib2

    @pl.when(b > 0)
    def _():
        out_ref[:, :] = out_ref[:, :] + contrib2


@functools.partial(jax.jit, static_argnames=("interpret",))
def kernel(x, gold, interpret=False):
    gold3 = gold.astype(jnp.int32).reshape(_B, 1, _S)
    out = pl.pallas_call(
        _loss_body,
        grid=(_B,),
        in_specs=[
            pl.BlockSpec((1, _S, _V), lambda b: (b, 0, 0)),
            pl.BlockSpec((1, 1, _S), lambda b: (b, 0, 0)),
        ],
        out_specs=pl.BlockSpec((1, 1), lambda b: (0, 0)),
        out_shape=jax.ShapeDtypeStruct((1, 1), jnp.float32),
        interpret=interpret,
    )(x, gold3)
    return out[0, 0]
